# Initial kernel scaffold; baseline (speedup 1.0000x reference)
import jax
import jax.numpy as jnp
from jax.experimental import pallas as pl


def kernel(x, edge_index, W1, a_src1, a_dst1, b1, W2, a_src2, a_dst2, b2, W3, a_src3, a_dst3, b3):
    raise NotImplementedError("write your pallas kernel here")


# speedup vs baseline: 25.7798x; 25.7798x over previous
import functools

import jax
import jax.numpy as jnp
from jax import lax
from jax.experimental import pallas as pl
from jax.experimental.pallas import tpu as pltpu
from jax.experimental.pallas import tpu_sc as plsc

N = 10000
E = 320000
D = 128
NP = 10240
NS = 16
K = 128
CH = 80
CB = 16
CAP = 32 * CH * K
ROWS_PT = NP // NS
F32 = jnp.float32


def _elu(g):
    return jnp.where(g > 0, g, jnp.exp(g) - 1.0)


def _tc_layer1_body(x_ref, w_ref, asrc_ref, adst_ref, h_ref, as_ref, ad_ref):
    h = jnp.dot(x_ref[...], w_ref[...], preferred_element_type=F32)
    h_ref[...] = h
    as_ref[...] = jnp.dot(h, asrc_ref[...], preferred_element_type=F32)
    ad_ref[...] = jnp.dot(h, adst_ref[...], preferred_element_type=F32)


def _tc_layer1(x, w, asrc, adst):
    grid = 8
    r = NP // grid
    return pl.pallas_call(
        _tc_layer1_body,
        grid=(grid,),
        in_specs=[
            pl.BlockSpec((r, D), lambda i: (i, 0)),
            pl.BlockSpec((D, D), lambda i: (0, 0)),
            pl.BlockSpec((D, 1), lambda i: (0, 0)),
            pl.BlockSpec((D, 1), lambda i: (0, 0)),
        ],
        out_specs=[
            pl.BlockSpec((r, D), lambda i: (i, 0)),
            pl.BlockSpec((r, 1), lambda i: (i, 0)),
            pl.BlockSpec((r, 1), lambda i: (i, 0)),
        ],
        out_shape=[
            jax.ShapeDtypeStruct((NP, D), F32),
            jax.ShapeDtypeStruct((NP, 1), F32),
            jax.ShapeDtypeStruct((NP, 1), F32),
        ],
    )(x, w, asrc, adst)


def _tc_layer2_body(na_ref, nb_ref, da_ref, db_ref, b_ref, w_ref, asrc_ref,
                    adst_ref, h_ref, as_ref, ad_ref):
    inv = 1.0 / (da_ref[...] + db_ref[...] + 1e-16)
    g = _elu((na_ref[...] + nb_ref[...]) * inv + b_ref[...])
    h = jnp.dot(g, w_ref[...], preferred_element_type=F32)
    h_ref[...] = h
    as_ref[...] = jnp.dot(h, asrc_ref[...], preferred_element_type=F32)
    ad_ref[...] = jnp.dot(h, adst_ref[...], preferred_element_type=F32)


def _tc_layer2(na, nb, da, db, b, w, asrc, adst):
    grid = 8
    r = NP // grid
    return pl.pallas_call(
        _tc_layer2_body,
        grid=(grid,),
        in_specs=[
            pl.BlockSpec((r, D), lambda i: (i, 0)),
            pl.BlockSpec((r, D), lambda i: (i, 0)),
            pl.BlockSpec((r, 1), lambda i: (i, 0)),
            pl.BlockSpec((r, 1), lambda i: (i, 0)),
            pl.BlockSpec((1, D), lambda i: (0, 0)),
            pl.BlockSpec((D, D), lambda i: (0, 0)),
            pl.BlockSpec((D, 1), lambda i: (0, 0)),
            pl.BlockSpec((D, 1), lambda i: (0, 0)),
        ],
        out_specs=[
            pl.BlockSpec((r, D), lambda i: (i, 0)),
            pl.BlockSpec((r, 1), lambda i: (i, 0)),
            pl.BlockSpec((r, 1), lambda i: (i, 0)),
        ],
        out_shape=[
            jax.ShapeDtypeStruct((NP, D), F32),
            jax.ShapeDtypeStruct((NP, 1), F32),
            jax.ShapeDtypeStruct((NP, 1), F32),
        ],
    )(na, nb, da, db, b, w, asrc, adst)


def _tc_layer3_body(na_ref, nb_ref, da_ref, db_ref, b_ref, w3_ref, ws_ref,
                    wd_ref, h3_ref, as_ref, ad_ref):
    inv = 1.0 / (da_ref[...] + db_ref[...] + 1e-16)
    g = _elu((na_ref[...] + nb_ref[...]) * inv + b_ref[...])
    h3_ref[...] = jnp.dot(g, w3_ref[...], preferred_element_type=F32)
    as_ref[...] = jnp.dot(g, ws_ref[...], preferred_element_type=F32)
    ad_ref[...] = jnp.dot(g, wd_ref[...], preferred_element_type=F32)


def _tc_layer3(na, nb, da, db, b, w3, ws, wd):
    grid = 8
    r = NP // grid
    return pl.pallas_call(
        _tc_layer3_body,
        grid=(grid,),
        in_specs=[
            pl.BlockSpec((r, D), lambda i: (i, 0)),
            pl.BlockSpec((r, D), lambda i: (i, 0)),
            pl.BlockSpec((r, 1), lambda i: (i, 0)),
            pl.BlockSpec((r, 1), lambda i: (i, 0)),
            pl.BlockSpec((1, D), lambda i: (0, 0)),
            pl.BlockSpec((D, 1), lambda i: (0, 0)),
            pl.BlockSpec((D, 1), lambda i: (0, 0)),
            pl.BlockSpec((D, 1), lambda i: (0, 0)),
        ],
        out_specs=[
            pl.BlockSpec((r, 1), lambda i: (i, 0)),
            pl.BlockSpec((r, 1), lambda i: (i, 0)),
            pl.BlockSpec((r, 1), lambda i: (i, 0)),
        ],
        out_shape=[
            jax.ShapeDtypeStruct((NP, 1), F32),
            jax.ShapeDtypeStruct((NP, 1), F32),
            jax.ShapeDtypeStruct((NP, 1), F32),
        ],
    )(na, nb, da, db, b, w3, ws, wd)


def _tc_logsoftmax_body(num_ref, den_ref, out_ref):
    n = num_ref[0:1, :] + num_ref[1:2, :]
    d = den_ref[0:1, :] + den_ref[1:2, :]
    l = n / (d + 1e-16)
    col = lax.broadcasted_iota(jnp.int32, (1, NP), 1)
    mask = col < N
    lm = jnp.where(mask, l, -3.0e38)
    m = jnp.max(lm)
    se = jnp.sum(jnp.where(mask, jnp.exp(lm - m), 0.0))
    out_ref[...] = (lm - m) - jnp.log(se)


def _tc_logsoftmax(num2, den2):
    return pl.pallas_call(
        _tc_logsoftmax_body,
        grid=(1,),
        in_specs=[
            pl.BlockSpec((2, NP), lambda i: (0, 0)),
            pl.BlockSpec((2, NP), lambda i: (0, 0)),
        ],
        out_specs=pl.BlockSpec((1, NP), lambda i: (0, 0)),
        out_shape=jax.ShapeDtypeStruct((1, NP), F32),
    )(num2, den2)


_MESH = plsc.VectorSubcoreMesh(core_axis_name="c", subcore_axis_name="s")


def _sc_edge12(h, asv, adv, src_t, dst_t, z2d, z1d):
    @functools.partial(
        pl.kernel,
        out_type=[
            jax.ShapeDtypeStruct((2 * NP, D), F32),
            jax.ShapeDtypeStruct((2 * NP,), F32),
        ],
        mesh=_MESH,
        compiler_params=pltpu.CompilerParams(needs_layout_passes=False),
        scratch_types=[
            pltpu.VMEM((CB, K), jnp.int32),
            pltpu.VMEM((CB, K), jnp.int32),
            pltpu.VMEM((NP,), F32),
            pltpu.VMEM((NP,), F32),
            pltpu.VMEM((K, D), F32),
            pltpu.VMEM((K,), F32),
            pltpu.VMEM_SHARED((NP, D), F32),
            pltpu.VMEM_SHARED((NP,), F32),
            pltpu.SemaphoreType.DMA,
        ],
    )
    def k(h_r, as_r, ad_r, src_r, dst_r, z2_r, z1_r,
          num_r, den_r,
          src_v, dst_v, as_v, ad_v, rows_v, exb, num_sh, den_sh, sem):
        c = lax.axis_index("c")
        s = lax.axis_index("s")
        w = c * NS + s
        r0 = s * ROWS_PT
        pltpu.sync_copy(z2_r.at[pl.ds(r0, ROWS_PT)],
                        num_sh.at[pl.ds(r0, ROWS_PT)])
        pltpu.sync_copy(z1_r.at[pl.ds(r0, ROWS_PT)],
                        den_sh.at[pl.ds(r0, ROWS_PT)])
        pltpu.sync_copy(as_r, as_v)
        pltpu.sync_copy(ad_r, ad_v)
        plsc.subcore_barrier()

        def body(i, carry):
            for j in range(K // 16):
                sidx = src_v[i, pl.ds(j * 16, 16)]
                didx = dst_v[i, pl.ds(j * 16, 16)]
                av = plsc.load_gather(as_v, [sidx])
                dv = plsc.load_gather(ad_v, [didx])
                e = av + dv
                e = jnp.where(e >= 0, e, 0.2 * e)
                exb[pl.ds(j * 16, 16)] = jnp.exp(e)
            pltpu.sync_copy(exb, den_sh.at[dst_v.at[i]], add=True)
            pltpu.async_copy(h_r.at[src_v.at[i]], rows_v, sem).wait()
            for g in range(K // 16):
                exv = exb[pl.ds(g * 16, 16)]
                for jj in range(16):
                    cj = exv[jj]
                    r = g * 16 + jj
                    for v in range(D // 16):
                        rows_v[r, pl.ds(v * 16, 16)] = (
                            rows_v[r, pl.ds(v * 16, 16)] * cj)
            pltpu.sync_copy(rows_v, num_sh.at[dst_v.at[i]], add=True)
            return carry

        def blk(bi, carry):
            pltpu.sync_copy(src_r.at[pl.ds(w * CH + bi * CB, CB)], src_v)
            pltpu.sync_copy(dst_r.at[pl.ds(w * CH + bi * CB, CB)], dst_v)
            lax.fori_loop(0, CB, body, 0)
            return carry

        lax.fori_loop(0, CH // CB, blk, 0)
        plsc.subcore_barrier()
        pltpu.sync_copy(num_sh.at[pl.ds(r0, ROWS_PT)],
                        num_r.at[pl.ds(c * NP + r0, ROWS_PT)])
        pltpu.sync_copy(den_sh.at[pl.ds(r0, ROWS_PT)],
                        den_r.at[pl.ds(c * NP + r0, ROWS_PT)])

    return k(h, asv, adv, src_t, dst_t, z2d, z1d)


def _sc_edge3(h3v, asv, adv, src_t, dst_t, z1d):
    @functools.partial(
        pl.kernel,
        out_type=[
            jax.ShapeDtypeStruct((2 * NP,), F32),
            jax.ShapeDtypeStruct((2 * NP,), F32),
        ],
        mesh=_MESH,
        compiler_params=pltpu.CompilerParams(needs_layout_passes=False),
        scratch_types=[
            pltpu.VMEM((CH, K), jnp.int32),
            pltpu.VMEM((CH, K), jnp.int32),
            pltpu.VMEM((NP,), F32),
            pltpu.VMEM((NP,), F32),
            pltpu.VMEM((NP,), F32),
            pltpu.VMEM((K,), F32),
            pltpu.VMEM((K,), F32),
            pltpu.VMEM_SHARED((NP,), F32),
            pltpu.VMEM_SHARED((NP,), F32),
        ],
    )
    def k(h3_r, as_r, ad_r, src_r, dst_r, z1_r,
          num_r, den_r,
          src_v, dst_v, h3_v, as_v, ad_v, exb, nb, num_sh, den_sh):
        c = lax.axis_index("c")
        s = lax.axis_index("s")
        w = c * NS + s
        r0 = s * ROWS_PT
        pltpu.sync_copy(z1_r.at[pl.ds(r0, ROWS_PT)],
                        num_sh.at[pl.ds(r0, ROWS_PT)])
        pltpu.sync_copy(z1_r.at[pl.ds(r0, ROWS_PT)],
                        den_sh.at[pl.ds(r0, ROWS_PT)])
        pltpu.sync_copy(src_r.at[pl.ds(w * CH, CH)], src_v)
        pltpu.sync_copy(dst_r.at[pl.ds(w * CH, CH)], dst_v)
        pltpu.sync_copy(h3_r, h3_v)
        pltpu.sync_copy(as_r, as_v)
        pltpu.sync_copy(ad_r, ad_v)
        plsc.subcore_barrier()

        def body(i, carry):
            for j in range(K // 16):
                sidx = src_v[i, pl.ds(j * 16, 16)]
                didx = dst_v[i, pl.ds(j * 16, 16)]
                av = plsc.load_gather(as_v, [sidx])
                dv = plsc.load_gather(ad_v, [didx])
                hv = plsc.load_gather(h3_v, [sidx])
                e = av + dv
                e = jnp.where(e >= 0, e, 0.2 * e)
                ex = jnp.exp(e)
                exb[pl.ds(j * 16, 16)] = ex
                nb[pl.ds(j * 16, 16)] = ex * hv
            pltpu.sync_copy(exb, den_sh.at[dst_v.at[i]], add=True)
            pltpu.sync_copy(nb, num_sh.at[dst_v.at[i]], add=True)
            return carry

        lax.fori_loop(0, CH, body, 0)
        plsc.subcore_barrier()
        pltpu.sync_copy(num_sh.at[pl.ds(r0, ROWS_PT)],
                        num_r.at[pl.ds(c * NP + r0, ROWS_PT)])
        pltpu.sync_copy(den_sh.at[pl.ds(r0, ROWS_PT)],
                        den_r.at[pl.ds(c * NP + r0, ROWS_PT)])

    return k(h3v, asv, adv, src_t, dst_t, z1d)


def kernel(x, edge_index, W1, a_src1, a_dst1, b1, W2, a_src2, a_dst2, b2,
           W3, a_src3, a_dst3, b3):
    src = edge_index[0].astype(jnp.int32)
    dst = edge_index[1].astype(jnp.int32)

    def pad_edges(v, fill):
        return jnp.concatenate(
            [v, jnp.full((CAP - E,), fill, jnp.int32)]).reshape(-1, K)

    src_t = pad_edges(src, 0)
    dst_t = pad_edges(dst, N)

    xp = jnp.pad(x, ((0, NP - N), (0, 0)))
    z2d = jnp.zeros((NP, D), F32)
    z1d = jnp.zeros((NP,), F32)

    def col(v):
        return v.reshape(D, 1)

    b1r = b1.reshape(1, D)
    b2r = b2.reshape(1, D)

    def split(num, den):
        num2 = num.reshape(2, NP, D)
        den2 = den.reshape(2, NP, 1)
        return num2[0], num2[1], den2[0], den2[1]

    h, asl, adl = _tc_layer1(xp, W1, col(a_src1), col(a_dst1))
    num, den = _sc_edge12(h, asl.reshape(NP), adl.reshape(NP),
                          src_t, dst_t, z2d, z1d)

    na, nbp, da, db = split(num, den)
    h, asl, adl = _tc_layer2(na, nbp, da, db, b1r, W2,
                             col(a_src2), col(a_dst2))
    num, den = _sc_edge12(h, asl.reshape(NP), adl.reshape(NP),
                          src_t, dst_t, z2d, z1d)

    na, nbp, da, db = split(num, den)
    w3c = W3.reshape(D, 1)
    h3, as3, ad3 = _tc_layer3(na, nbp, da, db, b2r, w3c,
                              w3c * a_src3[0], w3c * a_dst3[0])
    num3, den3 = _sc_edge3(h3.reshape(NP), as3.reshape(NP), ad3.reshape(NP),
                           src_t, dst_t, z1d)

    out = _tc_logsoftmax(num3.reshape(2, NP), den3.reshape(2, NP))
    return out[:, :N]


# speedup vs baseline: 31.0803x; 1.2056x over previous
import functools

import jax
import jax.numpy as jnp
from jax import lax
from jax.experimental import pallas as pl
from jax.experimental.pallas import tpu as pltpu
from jax.experimental.pallas import tpu_sc as plsc

N = 10000
E = 320000
D = 128
NP = 10240
NS = 16
K = 128
CH = 80
CB = 16
CAP = 32 * CH * K
ROWS_PT = NP // NS
F32 = jnp.float32


def _elu(g):
    return jnp.where(g > 0, g, jnp.exp(g) - 1.0)


def _tc_layer1_body(x_ref, w_ref, asrc_ref, adst_ref, h_ref, as_ref, ad_ref):
    h = jnp.dot(x_ref[...], w_ref[...], preferred_element_type=F32)
    h_ref[...] = h
    as_ref[...] = jnp.dot(h, asrc_ref[...], preferred_element_type=F32)
    ad_ref[...] = jnp.dot(h, adst_ref[...], preferred_element_type=F32)


def _tc_layer1(x, w, asrc, adst):
    grid = 8
    r = NP // grid
    return pl.pallas_call(
        _tc_layer1_body,
        grid=(grid,),
        in_specs=[
            pl.BlockSpec((r, D), lambda i: (i, 0)),
            pl.BlockSpec((D, D), lambda i: (0, 0)),
            pl.BlockSpec((D, 1), lambda i: (0, 0)),
            pl.BlockSpec((D, 1), lambda i: (0, 0)),
        ],
        out_specs=[
            pl.BlockSpec((r, D), lambda i: (i, 0)),
            pl.BlockSpec((r, 1), lambda i: (i, 0)),
            pl.BlockSpec((r, 1), lambda i: (i, 0)),
        ],
        out_shape=[
            jax.ShapeDtypeStruct((NP, D), F32),
            jax.ShapeDtypeStruct((NP, 1), F32),
            jax.ShapeDtypeStruct((NP, 1), F32),
        ],
    )(x, w, asrc, adst)


def _tc_layer2_body(na_ref, nb_ref, da_ref, db_ref, b_ref, w_ref, asrc_ref,
                    adst_ref, h_ref, as_ref, ad_ref):
    inv = 1.0 / (da_ref[...] + db_ref[...] + 1e-16)
    g = _elu((na_ref[...] + nb_ref[...]) * inv + b_ref[...])
    h = jnp.dot(g, w_ref[...], preferred_element_type=F32)
    h_ref[...] = h
    as_ref[...] = jnp.dot(h, asrc_ref[...], preferred_element_type=F32)
    ad_ref[...] = jnp.dot(h, adst_ref[...], preferred_element_type=F32)


def _tc_layer2(na, nb, da, db, b, w, asrc, adst):
    grid = 8
    r = NP // grid
    return pl.pallas_call(
        _tc_layer2_body,
        grid=(grid,),
        in_specs=[
            pl.BlockSpec((r, D), lambda i: (i, 0)),
            pl.BlockSpec((r, D), lambda i: (i, 0)),
            pl.BlockSpec((r, 1), lambda i: (i, 0)),
            pl.BlockSpec((r, 1), lambda i: (i, 0)),
            pl.BlockSpec((1, D), lambda i: (0, 0)),
            pl.BlockSpec((D, D), lambda i: (0, 0)),
            pl.BlockSpec((D, 1), lambda i: (0, 0)),
            pl.BlockSpec((D, 1), lambda i: (0, 0)),
        ],
        out_specs=[
            pl.BlockSpec((r, D), lambda i: (i, 0)),
            pl.BlockSpec((r, 1), lambda i: (i, 0)),
            pl.BlockSpec((r, 1), lambda i: (i, 0)),
        ],
        out_shape=[
            jax.ShapeDtypeStruct((NP, D), F32),
            jax.ShapeDtypeStruct((NP, 1), F32),
            jax.ShapeDtypeStruct((NP, 1), F32),
        ],
    )(na, nb, da, db, b, w, asrc, adst)


def _tc_layer3_body(na_ref, nb_ref, da_ref, db_ref, b_ref, w3_ref, ws_ref,
                    wd_ref, h3_ref, as_ref, ad_ref):
    inv = 1.0 / (da_ref[...] + db_ref[...] + 1e-16)
    g = _elu((na_ref[...] + nb_ref[...]) * inv + b_ref[...])
    h3_ref[...] = jnp.dot(g, w3_ref[...], preferred_element_type=F32)
    as_ref[...] = jnp.dot(g, ws_ref[...], preferred_element_type=F32)
    ad_ref[...] = jnp.dot(g, wd_ref[...], preferred_element_type=F32)


def _tc_layer3(na, nb, da, db, b, w3, ws, wd):
    grid = 8
    r = NP // grid
    return pl.pallas_call(
        _tc_layer3_body,
        grid=(grid,),
        in_specs=[
            pl.BlockSpec((r, D), lambda i: (i, 0)),
            pl.BlockSpec((r, D), lambda i: (i, 0)),
            pl.BlockSpec((r, 1), lambda i: (i, 0)),
            pl.BlockSpec((r, 1), lambda i: (i, 0)),
            pl.BlockSpec((1, D), lambda i: (0, 0)),
            pl.BlockSpec((D, 1), lambda i: (0, 0)),
            pl.BlockSpec((D, 1), lambda i: (0, 0)),
            pl.BlockSpec((D, 1), lambda i: (0, 0)),
        ],
        out_specs=[
            pl.BlockSpec((r, 1), lambda i: (i, 0)),
            pl.BlockSpec((r, 1), lambda i: (i, 0)),
            pl.BlockSpec((r, 1), lambda i: (i, 0)),
        ],
        out_shape=[
            jax.ShapeDtypeStruct((NP, 1), F32),
            jax.ShapeDtypeStruct((NP, 1), F32),
            jax.ShapeDtypeStruct((NP, 1), F32),
        ],
    )(na, nb, da, db, b, w3, ws, wd)


def _tc_logsoftmax_body(num_ref, den_ref, out_ref):
    n = num_ref[0:1, :] + num_ref[1:2, :]
    d = den_ref[0:1, :] + den_ref[1:2, :]
    l = n / (d + 1e-16)
    col = lax.broadcasted_iota(jnp.int32, (1, NP), 1)
    mask = col < N
    lm = jnp.where(mask, l, -3.0e38)
    m = jnp.max(lm)
    se = jnp.sum(jnp.where(mask, jnp.exp(lm - m), 0.0))
    out_ref[...] = (lm - m) - jnp.log(se)


def _tc_logsoftmax(num2, den2):
    return pl.pallas_call(
        _tc_logsoftmax_body,
        grid=(1,),
        in_specs=[
            pl.BlockSpec((2, NP), lambda i: (0, 0)),
            pl.BlockSpec((2, NP), lambda i: (0, 0)),
        ],
        out_specs=pl.BlockSpec((1, NP), lambda i: (0, 0)),
        out_shape=jax.ShapeDtypeStruct((1, NP), F32),
    )(num2, den2)


_MESH = plsc.VectorSubcoreMesh(core_axis_name="c", subcore_axis_name="s")


def _sc_edge12(h, asv, adv, src_t, dst_t, z2d, z1d):
    @functools.partial(
        pl.kernel,
        out_type=[
            jax.ShapeDtypeStruct((2 * NP, D), F32),
            jax.ShapeDtypeStruct((2 * NP,), F32),
        ],
        mesh=_MESH,
        compiler_params=pltpu.CompilerParams(needs_layout_passes=False),
        scratch_types=[
            pltpu.VMEM((CB, K), jnp.int32),
            pltpu.VMEM((CB, K), jnp.int32),
            pltpu.VMEM((K, D), F32),
            pltpu.VMEM((K, D), F32),
            pltpu.VMEM((K,), F32),
            pltpu.VMEM((K,), F32),
            pltpu.VMEM((K,), F32),
            pltpu.VMEM((K,), F32),
            pltpu.VMEM((K,), F32),
            pltpu.VMEM_SHARED((NP,), F32),
            pltpu.VMEM_SHARED((NP,), F32),
            pltpu.VMEM_SHARED((NP, D), F32),
            pltpu.VMEM_SHARED((NP,), F32),
            pltpu.SemaphoreType.DMA,
            pltpu.SemaphoreType.DMA,
            pltpu.SemaphoreType.DMA,
            pltpu.SemaphoreType.DMA,
        ],
    )
    def k(h_r, as_r, ad_r, src_r, dst_r, z2_r, z1_r,
          num_r, den_r,
          src_v, dst_v, rows_v0, rows_v1, asb0, asb1, adb0, adb1, exb,
          as_sh, ad_sh, num_sh, den_sh,
          sem0, sem1, semA0, semA1):
        c = lax.axis_index("c")
        s = lax.axis_index("s")
        w = c * NS + s
        r0 = s * ROWS_PT
        pltpu.sync_copy(z2_r.at[pl.ds(r0, ROWS_PT)],
                        num_sh.at[pl.ds(r0, ROWS_PT)])
        pltpu.sync_copy(z1_r.at[pl.ds(r0, ROWS_PT)],
                        den_sh.at[pl.ds(r0, ROWS_PT)])
        pltpu.sync_copy(as_r.at[pl.ds(r0, ROWS_PT)],
                        as_sh.at[pl.ds(r0, ROWS_PT)])
        pltpu.sync_copy(ad_r.at[pl.ds(r0, ROWS_PT)],
                        ad_sh.at[pl.ds(r0, ROWS_PT)])
        plsc.subcore_barrier()

        def fetch(i, rows_v, asb, adb, sem, semA):
            pltpu.async_copy(h_r.at[src_v.at[i]], rows_v, sem)
            pltpu.async_copy(as_sh.at[src_v.at[i]], asb, semA)
            pltpu.async_copy(ad_sh.at[dst_v.at[i]], adb, semA)

        def chunk(i, rows_v, asb, adb, sem, semA,
                  rows_o, asb_o, adb_o, sem_o, semA_o, prefetch):
            @pl.when(prefetch)
            def _():
                fetch(i + 1, rows_o, asb_o, adb_o, sem_o, semA_o)

            pltpu.make_async_copy(as_sh.at[src_v.at[i]], asb, semA).wait()
            pltpu.make_async_copy(ad_sh.at[dst_v.at[i]], adb, semA).wait()
            for j in range(K // 16):
                e = asb[pl.ds(j * 16, 16)] + adb[pl.ds(j * 16, 16)]
                e = jnp.where(e >= 0, e, 0.2 * e)
                exb[pl.ds(j * 16, 16)] = jnp.exp(e)
            pltpu.sync_copy(exb, den_sh.at[dst_v.at[i]], add=True)
            pltpu.make_async_copy(h_r.at[src_v.at[i]], rows_v, sem).wait()
            for g in range(K // 16):
                exv = exb[pl.ds(g * 16, 16)]
                for jj in range(16):
                    cj = exv[jj]
                    r = g * 16 + jj
                    for v in range(D // 16):
                        rows_v[r, pl.ds(v * 16, 16)] = (
                            rows_v[r, pl.ds(v * 16, 16)] * cj)
            pltpu.sync_copy(rows_v, num_sh.at[dst_v.at[i]], add=True)

        def blk(bi, carry):
            pltpu.sync_copy(src_r.at[pl.ds(w * CH + bi * CB, CB)], src_v)
            pltpu.sync_copy(dst_r.at[pl.ds(w * CH + bi * CB, CB)], dst_v)
            fetch(0, rows_v0, asb0, adb0, sem0, semA0)

            def pair(g, c2):
                i = 2 * g
                chunk(i, rows_v0, asb0, adb0, sem0, semA0,
                      rows_v1, asb1, adb1, sem1, semA1, True)
                chunk(i + 1, rows_v1, asb1, adb1, sem1, semA1,
                      rows_v0, asb0, adb0, sem0, semA0, g < CB // 2 - 1)
                return c2

            lax.fori_loop(0, CB // 2, pair, 0)
            return carry

        lax.fori_loop(0, CH // CB, blk, 0)
        plsc.subcore_barrier()
        pltpu.sync_copy(num_sh.at[pl.ds(r0, ROWS_PT)],
                        num_r.at[pl.ds(c * NP + r0, ROWS_PT)])
        pltpu.sync_copy(den_sh.at[pl.ds(r0, ROWS_PT)],
                        den_r.at[pl.ds(c * NP + r0, ROWS_PT)])

    return k(h, asv, adv, src_t, dst_t, z2d, z1d)


def _sc_edge3(h3v, asv, adv, src_t, dst_t, z1d):
    @functools.partial(
        pl.kernel,
        out_type=[
            jax.ShapeDtypeStruct((2 * NP,), F32),
            jax.ShapeDtypeStruct((2 * NP,), F32),
        ],
        mesh=_MESH,
        compiler_params=pltpu.CompilerParams(needs_layout_passes=False),
        scratch_types=[
            pltpu.VMEM((CH, K), jnp.int32),
            pltpu.VMEM((CH, K), jnp.int32),
            pltpu.VMEM((NP,), F32),
            pltpu.VMEM((NP,), F32),
            pltpu.VMEM((NP,), F32),
            pltpu.VMEM((K,), F32),
            pltpu.VMEM((K,), F32),
            pltpu.VMEM_SHARED((NP,), F32),
            pltpu.VMEM_SHARED((NP,), F32),
        ],
    )
    def k(h3_r, as_r, ad_r, src_r, dst_r, z1_r,
          num_r, den_r,
          src_v, dst_v, h3_v, as_v, ad_v, exb, nb, num_sh, den_sh):
        c = lax.axis_index("c")
        s = lax.axis_index("s")
        w = c * NS + s
        r0 = s * ROWS_PT
        pltpu.sync_copy(z1_r.at[pl.ds(r0, ROWS_PT)],
                        num_sh.at[pl.ds(r0, ROWS_PT)])
        pltpu.sync_copy(z1_r.at[pl.ds(r0, ROWS_PT)],
                        den_sh.at[pl.ds(r0, ROWS_PT)])
        pltpu.sync_copy(src_r.at[pl.ds(w * CH, CH)], src_v)
        pltpu.sync_copy(dst_r.at[pl.ds(w * CH, CH)], dst_v)
        pltpu.sync_copy(h3_r, h3_v)
        pltpu.sync_copy(as_r, as_v)
        pltpu.sync_copy(ad_r, ad_v)
        plsc.subcore_barrier()

        def body(i, carry):
            for j in range(K // 16):
                sidx = src_v[i, pl.ds(j * 16, 16)]
                didx = dst_v[i, pl.ds(j * 16, 16)]
                av = plsc.load_gather(as_v, [sidx])
                dv = plsc.load_gather(ad_v, [didx])
                hv = plsc.load_gather(h3_v, [sidx])
                e = av + dv
                e = jnp.where(e >= 0, e, 0.2 * e)
                ex = jnp.exp(e)
                exb[pl.ds(j * 16, 16)] = ex
                nb[pl.ds(j * 16, 16)] = ex * hv
            pltpu.sync_copy(exb, den_sh.at[dst_v.at[i]], add=True)
            pltpu.sync_copy(nb, num_sh.at[dst_v.at[i]], add=True)
            return carry

        lax.fori_loop(0, CH, body, 0)
        plsc.subcore_barrier()
        pltpu.sync_copy(num_sh.at[pl.ds(r0, ROWS_PT)],
                        num_r.at[pl.ds(c * NP + r0, ROWS_PT)])
        pltpu.sync_copy(den_sh.at[pl.ds(r0, ROWS_PT)],
                        den_r.at[pl.ds(c * NP + r0, ROWS_PT)])

    return k(h3v, asv, adv, src_t, dst_t, z1d)


def kernel(x, edge_index, W1, a_src1, a_dst1, b1, W2, a_src2, a_dst2, b2,
           W3, a_src3, a_dst3, b3):
    src = edge_index[0].astype(jnp.int32)
    dst = edge_index[1].astype(jnp.int32)

    def pad_edges(v, fill):
        return jnp.concatenate(
            [v, jnp.full((CAP - E,), fill, jnp.int32)]).reshape(-1, K)

    src_t = pad_edges(src, 0)
    dst_t = pad_edges(dst, N)

    xp = jnp.pad(x, ((0, NP - N), (0, 0)))
    z2d = jnp.zeros((NP, D), F32)
    z1d = jnp.zeros((NP,), F32)

    def col(v):
        return v.reshape(D, 1)

    b1r = b1.reshape(1, D)
    b2r = b2.reshape(1, D)

    def split(num, den):
        num2 = num.reshape(2, NP, D)
        den2 = den.reshape(2, NP, 1)
        return num2[0], num2[1], den2[0], den2[1]

    h, asl, adl = _tc_layer1(xp, W1, col(a_src1), col(a_dst1))
    num, den = _sc_edge12(h, asl.reshape(NP), adl.reshape(NP),
                          src_t, dst_t, z2d, z1d)

    na, nbp, da, db = split(num, den)
    h, asl, adl = _tc_layer2(na, nbp, da, db, b1r, W2,
                             col(a_src2), col(a_dst2))
    num, den = _sc_edge12(h, asl.reshape(NP), adl.reshape(NP),
                          src_t, dst_t, z2d, z1d)

    na, nbp, da, db = split(num, den)
    w3c = W3.reshape(D, 1)
    h3, as3, ad3 = _tc_layer3(na, nbp, da, db, b2r, w3c,
                              w3c * a_src3[0], w3c * a_dst3[0])
    num3, den3 = _sc_edge3(h3.reshape(NP), as3.reshape(NP), ad3.reshape(NP),
                           src_t, dst_t, z1d)

    out = _tc_logsoftmax(num3.reshape(2, NP), den3.reshape(2, NP))
    return out[:, :N]


# speedup vs baseline: 31.3887x; 1.0099x over previous
import functools

import jax
import jax.numpy as jnp
from jax import lax
from jax.experimental import pallas as pl
from jax.experimental.pallas import tpu as pltpu
from jax.experimental.pallas import tpu_sc as plsc

N = 10000
E = 320000
D = 128
NP = 10240
NS = 16
K = 128
CH = 80
CB = 16
CAP = 32 * CH * K
ROWS_PT = NP // NS
F32 = jnp.float32


def _elu(g):
    return jnp.where(g > 0, g, jnp.exp(g) - 1.0)


def _tc_layer1_body(x_ref, w_ref, asrc_ref, adst_ref, h_ref, as_ref, ad_ref):
    h = jnp.dot(x_ref[...], w_ref[...], preferred_element_type=F32)
    h_ref[...] = h
    as_ref[...] = jnp.dot(h, asrc_ref[...], preferred_element_type=F32)
    ad_ref[...] = jnp.dot(h, adst_ref[...], preferred_element_type=F32)


def _tc_layer1(x, w, asrc, adst):
    grid = 8
    r = NP // grid
    return pl.pallas_call(
        _tc_layer1_body,
        grid=(grid,),
        in_specs=[
            pl.BlockSpec((r, D), lambda i: (i, 0)),
            pl.BlockSpec((D, D), lambda i: (0, 0)),
            pl.BlockSpec((D, 1), lambda i: (0, 0)),
            pl.BlockSpec((D, 1), lambda i: (0, 0)),
        ],
        out_specs=[
            pl.BlockSpec((r, D), lambda i: (i, 0)),
            pl.BlockSpec((r, 1), lambda i: (i, 0)),
            pl.BlockSpec((r, 1), lambda i: (i, 0)),
        ],
        out_shape=[
            jax.ShapeDtypeStruct((NP, D), F32),
            jax.ShapeDtypeStruct((NP, 1), F32),
            jax.ShapeDtypeStruct((NP, 1), F32),
        ],
    )(x, w, asrc, adst)


def _tc_layer2_body(na_ref, nb_ref, da_ref, db_ref, b_ref, w_ref, asrc_ref,
                    adst_ref, h_ref, as_ref, ad_ref):
    inv = 1.0 / (da_ref[...] + db_ref[...] + 1e-16)
    g = _elu((na_ref[...] + nb_ref[...]) * inv + b_ref[...])
    h = jnp.dot(g, w_ref[...], preferred_element_type=F32)
    h_ref[...] = h
    as_ref[...] = jnp.dot(h, asrc_ref[...], preferred_element_type=F32)
    ad_ref[...] = jnp.dot(h, adst_ref[...], preferred_element_type=F32)


def _tc_layer2(na, nb, da, db, b, w, asrc, adst):
    grid = 8
    r = NP // grid
    return pl.pallas_call(
        _tc_layer2_body,
        grid=(grid,),
        in_specs=[
            pl.BlockSpec((r, D), lambda i: (i, 0)),
            pl.BlockSpec((r, D), lambda i: (i, 0)),
            pl.BlockSpec((r, 1), lambda i: (i, 0)),
            pl.BlockSpec((r, 1), lambda i: (i, 0)),
            pl.BlockSpec((1, D), lambda i: (0, 0)),
            pl.BlockSpec((D, D), lambda i: (0, 0)),
            pl.BlockSpec((D, 1), lambda i: (0, 0)),
            pl.BlockSpec((D, 1), lambda i: (0, 0)),
        ],
        out_specs=[
            pl.BlockSpec((r, D), lambda i: (i, 0)),
            pl.BlockSpec((r, 1), lambda i: (i, 0)),
            pl.BlockSpec((r, 1), lambda i: (i, 0)),
        ],
        out_shape=[
            jax.ShapeDtypeStruct((NP, D), F32),
            jax.ShapeDtypeStruct((NP, 1), F32),
            jax.ShapeDtypeStruct((NP, 1), F32),
        ],
    )(na, nb, da, db, b, w, asrc, adst)


def _tc_layer3_body(na_ref, nb_ref, da_ref, db_ref, b_ref, w3_ref, ws_ref,
                    wd_ref, h3_ref, as_ref, ad_ref):
    inv = 1.0 / (da_ref[...] + db_ref[...] + 1e-16)
    g = _elu((na_ref[...] + nb_ref[...]) * inv + b_ref[...])
    h3_ref[...] = jnp.dot(g, w3_ref[...], preferred_element_type=F32)
    as_ref[...] = jnp.dot(g, ws_ref[...], preferred_element_type=F32)
    ad_ref[...] = jnp.dot(g, wd_ref[...], preferred_element_type=F32)


def _tc_layer3(na, nb, da, db, b, w3, ws, wd):
    grid = 8
    r = NP // grid
    return pl.pallas_call(
        _tc_layer3_body,
        grid=(grid,),
        in_specs=[
            pl.BlockSpec((r, D), lambda i: (i, 0)),
            pl.BlockSpec((r, D), lambda i: (i, 0)),
            pl.BlockSpec((r, 1), lambda i: (i, 0)),
            pl.BlockSpec((r, 1), lambda i: (i, 0)),
            pl.BlockSpec((1, D), lambda i: (0, 0)),
            pl.BlockSpec((D, 1), lambda i: (0, 0)),
            pl.BlockSpec((D, 1), lambda i: (0, 0)),
            pl.BlockSpec((D, 1), lambda i: (0, 0)),
        ],
        out_specs=[
            pl.BlockSpec((r, 1), lambda i: (i, 0)),
            pl.BlockSpec((r, 1), lambda i: (i, 0)),
            pl.BlockSpec((r, 1), lambda i: (i, 0)),
        ],
        out_shape=[
            jax.ShapeDtypeStruct((NP, 1), F32),
            jax.ShapeDtypeStruct((NP, 1), F32),
            jax.ShapeDtypeStruct((NP, 1), F32),
        ],
    )(na, nb, da, db, b, w3, ws, wd)


def _tc_logsoftmax_body(num_ref, den_ref, out_ref):
    n = num_ref[0:1, :] + num_ref[1:2, :]
    d = den_ref[0:1, :] + den_ref[1:2, :]
    l = n / (d + 1e-16)
    col = lax.broadcasted_iota(jnp.int32, (1, NP), 1)
    mask = col < N
    lm = jnp.where(mask, l, -3.0e38)
    m = jnp.max(lm)
    se = jnp.sum(jnp.where(mask, jnp.exp(lm - m), 0.0))
    out_ref[...] = (lm - m) - jnp.log(se)


def _tc_logsoftmax(num2, den2):
    return pl.pallas_call(
        _tc_logsoftmax_body,
        grid=(1,),
        in_specs=[
            pl.BlockSpec((2, NP), lambda i: (0, 0)),
            pl.BlockSpec((2, NP), lambda i: (0, 0)),
        ],
        out_specs=pl.BlockSpec((1, NP), lambda i: (0, 0)),
        out_shape=jax.ShapeDtypeStruct((1, NP), F32),
    )(num2, den2)


_MESH = plsc.VectorSubcoreMesh(core_axis_name="c", subcore_axis_name="s")


def _sc_edge12(h, asv, adv, src_t, dst_t, z2d, z1d):
    @functools.partial(
        pl.kernel,
        out_type=[
            jax.ShapeDtypeStruct((2 * NP, D), F32),
            jax.ShapeDtypeStruct((2 * NP,), F32),
        ],
        mesh=_MESH,
        compiler_params=pltpu.CompilerParams(needs_layout_passes=False),
        scratch_types=[
            pltpu.VMEM((CB, K), jnp.int32),
            pltpu.VMEM((CB, K), jnp.int32),
            pltpu.VMEM((K, D), F32),
            pltpu.VMEM((K, D), F32),
            pltpu.VMEM((K,), F32),
            pltpu.VMEM((K,), F32),
            pltpu.VMEM((K,), F32),
            pltpu.VMEM((K,), F32),
            pltpu.VMEM((K,), F32),
            pltpu.VMEM((K,), F32),
            pltpu.VMEM_SHARED((NP,), F32),
            pltpu.VMEM_SHARED((NP,), F32),
            pltpu.VMEM_SHARED((NP, D), F32),
            pltpu.VMEM_SHARED((NP,), F32),
            pltpu.SemaphoreType.DMA,
            pltpu.SemaphoreType.DMA,
            pltpu.SemaphoreType.DMA,
            pltpu.SemaphoreType.DMA,
            pltpu.SemaphoreType.DMA,
            pltpu.SemaphoreType.DMA,
            pltpu.SemaphoreType.DMA,
            pltpu.SemaphoreType.DMA,
        ],
    )
    def k(h_r, as_r, ad_r, src_r, dst_r, z2_r, z1_r,
          num_r, den_r,
          src_v, dst_v, rows_v0, rows_v1, asb0, asb1, adb0, adb1, exb0, exb1,
          as_sh, ad_sh, num_sh, den_sh,
          sem0, sem1, semA0, semA1, semD0, semD1, semS0, semS1):
        c = lax.axis_index("c")
        s = lax.axis_index("s")
        w = c * NS + s
        r0 = s * ROWS_PT
        pltpu.sync_copy(z2_r.at[pl.ds(r0, ROWS_PT)],
                        num_sh.at[pl.ds(r0, ROWS_PT)])
        pltpu.sync_copy(z1_r.at[pl.ds(r0, ROWS_PT)],
                        den_sh.at[pl.ds(r0, ROWS_PT)])
        pltpu.sync_copy(as_r.at[pl.ds(r0, ROWS_PT)],
                        as_sh.at[pl.ds(r0, ROWS_PT)])
        pltpu.sync_copy(ad_r.at[pl.ds(r0, ROWS_PT)],
                        ad_sh.at[pl.ds(r0, ROWS_PT)])
        plsc.subcore_barrier()

        def fetch(i, rows_v, asb, adb, sem, semA):
            pltpu.async_copy(h_r.at[src_v.at[i]], rows_v, sem)
            pltpu.async_copy(as_sh.at[src_v.at[i]], asb, semA)
            pltpu.async_copy(ad_sh.at[dst_v.at[i]], adb, semA)

        def chunk(i, rows_v, asb, adb, exb, sem, semA, semD, semS,
                  rows_o, asb_o, adb_o, sem_o, semA_o, semS_o,
                  prefetch, pend_d, pend_s):
            @pl.when(pend_d)
            def _():
                pltpu.make_async_copy(exb, den_sh.at[dst_v.at[i]],
                                      semD).wait()
            pltpu.make_async_copy(as_sh.at[src_v.at[i]], asb, semA).wait()
            pltpu.make_async_copy(ad_sh.at[dst_v.at[i]], adb, semA).wait()
            for j in range(K // 16):
                e = asb[pl.ds(j * 16, 16)] + adb[pl.ds(j * 16, 16)]
                e = jnp.where(e >= 0, e, 0.2 * e)
                exb[pl.ds(j * 16, 16)] = jnp.exp(e)
            pltpu.async_copy(exb, den_sh.at[dst_v.at[i]], semD, add=True)

            @pl.when(pend_s)
            def _():
                pltpu.make_async_copy(rows_o, num_sh.at[dst_v.at[i]],
                                      semS_o).wait()

            @pl.when(prefetch)
            def _():
                fetch(i + 1, rows_o, asb_o, adb_o, sem_o, semA_o)

            pltpu.make_async_copy(h_r.at[src_v.at[i]], rows_v, sem).wait()
            for g in range(K // 16):
                exv = exb[pl.ds(g * 16, 16)]
                for jj in range(16):
                    cj = exv[jj]
                    r = g * 16 + jj
                    for v in range(D // 16):
                        rows_v[r, pl.ds(v * 16, 16)] = (
                            rows_v[r, pl.ds(v * 16, 16)] * cj)
            pltpu.async_copy(rows_v, num_sh.at[dst_v.at[i]], semS, add=True)

        def blk(bi, carry):
            pltpu.sync_copy(src_r.at[pl.ds(w * CH + bi * CB, CB)], src_v)
            pltpu.sync_copy(dst_r.at[pl.ds(w * CH + bi * CB, CB)], dst_v)
            fetch(0, rows_v0, asb0, adb0, sem0, semA0)

            def pair(g, c2):
                i = 2 * g
                ge1 = g >= 1
                chunk(i, rows_v0, asb0, adb0, exb0, sem0, semA0, semD0, semS0,
                      rows_v1, asb1, adb1, sem1, semA1, semS1,
                      True, ge1, ge1)
                chunk(i + 1, rows_v1, asb1, adb1, exb1, sem1, semA1, semD1,
                      semS1, rows_v0, asb0, adb0, sem0, semA0, semS0,
                      g < CB // 2 - 1, ge1, True)
                return c2

            lax.fori_loop(0, CB // 2, pair, 0)
            pltpu.make_async_copy(rows_v1, num_sh.at[dst_v.at[CB - 1]],
                                  semS1).wait()
            pltpu.make_async_copy(exb0, den_sh.at[dst_v.at[CB - 2]],
                                  semD0).wait()
            pltpu.make_async_copy(exb1, den_sh.at[dst_v.at[CB - 1]],
                                  semD1).wait()
            return carry

        lax.fori_loop(0, CH // CB, blk, 0)
        plsc.subcore_barrier()
        pltpu.sync_copy(num_sh.at[pl.ds(r0, ROWS_PT)],
                        num_r.at[pl.ds(c * NP + r0, ROWS_PT)])
        pltpu.sync_copy(den_sh.at[pl.ds(r0, ROWS_PT)],
                        den_r.at[pl.ds(c * NP + r0, ROWS_PT)])

    return k(h, asv, adv, src_t, dst_t, z2d, z1d)


def _sc_edge3(h3v, asv, adv, src_t, dst_t, z1d):
    @functools.partial(
        pl.kernel,
        out_type=[
            jax.ShapeDtypeStruct((2 * NP,), F32),
            jax.ShapeDtypeStruct((2 * NP,), F32),
        ],
        mesh=_MESH,
        compiler_params=pltpu.CompilerParams(needs_layout_passes=False),
        scratch_types=[
            pltpu.VMEM((CH, K), jnp.int32),
            pltpu.VMEM((CH, K), jnp.int32),
            pltpu.VMEM((NP,), F32),
            pltpu.VMEM((NP,), F32),
            pltpu.VMEM((NP,), F32),
            pltpu.VMEM((K,), F32),
            pltpu.VMEM((K,), F32),
            pltpu.VMEM_SHARED((NP,), F32),
            pltpu.VMEM_SHARED((NP,), F32),
        ],
    )
    def k(h3_r, as_r, ad_r, src_r, dst_r, z1_r,
          num_r, den_r,
          src_v, dst_v, h3_v, as_v, ad_v, exb, nb, num_sh, den_sh):
        c = lax.axis_index("c")
        s = lax.axis_index("s")
        w = c * NS + s
        r0 = s * ROWS_PT
        pltpu.sync_copy(z1_r.at[pl.ds(r0, ROWS_PT)],
                        num_sh.at[pl.ds(r0, ROWS_PT)])
        pltpu.sync_copy(z1_r.at[pl.ds(r0, ROWS_PT)],
                        den_sh.at[pl.ds(r0, ROWS_PT)])
        pltpu.sync_copy(src_r.at[pl.ds(w * CH, CH)], src_v)
        pltpu.sync_copy(dst_r.at[pl.ds(w * CH, CH)], dst_v)
        pltpu.sync_copy(h3_r, h3_v)
        pltpu.sync_copy(as_r, as_v)
        pltpu.sync_copy(ad_r, ad_v)
        plsc.subcore_barrier()

        def body(i, carry):
            for j in range(K // 16):
                sidx = src_v[i, pl.ds(j * 16, 16)]
                didx = dst_v[i, pl.ds(j * 16, 16)]
                av = plsc.load_gather(as_v, [sidx])
                dv = plsc.load_gather(ad_v, [didx])
                hv = plsc.load_gather(h3_v, [sidx])
                e = av + dv
                e = jnp.where(e >= 0, e, 0.2 * e)
                ex = jnp.exp(e)
                exb[pl.ds(j * 16, 16)] = ex
                nb[pl.ds(j * 16, 16)] = ex * hv
            pltpu.sync_copy(exb, den_sh.at[dst_v.at[i]], add=True)
            pltpu.sync_copy(nb, num_sh.at[dst_v.at[i]], add=True)
            return carry

        lax.fori_loop(0, CH, body, 0)
        plsc.subcore_barrier()
        pltpu.sync_copy(num_sh.at[pl.ds(r0, ROWS_PT)],
                        num_r.at[pl.ds(c * NP + r0, ROWS_PT)])
        pltpu.sync_copy(den_sh.at[pl.ds(r0, ROWS_PT)],
                        den_r.at[pl.ds(c * NP + r0, ROWS_PT)])

    return k(h3v, asv, adv, src_t, dst_t, z1d)


def kernel(x, edge_index, W1, a_src1, a_dst1, b1, W2, a_src2, a_dst2, b2,
           W3, a_src3, a_dst3, b3):
    src = edge_index[0].astype(jnp.int32)
    dst = edge_index[1].astype(jnp.int32)

    def pad_edges(v, fill):
        return jnp.concatenate(
            [v, jnp.full((CAP - E,), fill, jnp.int32)]).reshape(-1, K)

    src_t = pad_edges(src, 0)
    dst_t = pad_edges(dst, N)

    xp = jnp.pad(x, ((0, NP - N), (0, 0)))
    z2d = jnp.zeros((NP, D), F32)
    z1d = jnp.zeros((NP,), F32)

    def col(v):
        return v.reshape(D, 1)

    b1r = b1.reshape(1, D)
    b2r = b2.reshape(1, D)

    def split(num, den):
        num2 = num.reshape(2, NP, D)
        den2 = den.reshape(2, NP, 1)
        return num2[0], num2[1], den2[0], den2[1]

    h, asl, adl = _tc_layer1(xp, W1, col(a_src1), col(a_dst1))
    num, den = _sc_edge12(h, asl.reshape(NP), adl.reshape(NP),
                          src_t, dst_t, z2d, z1d)

    na, nbp, da, db = split(num, den)
    h, asl, adl = _tc_layer2(na, nbp, da, db, b1r, W2,
                             col(a_src2), col(a_dst2))
    num, den = _sc_edge12(h, asl.reshape(NP), adl.reshape(NP),
                          src_t, dst_t, z2d, z1d)

    na, nbp, da, db = split(num, den)
    w3c = W3.reshape(D, 1)
    h3, as3, ad3 = _tc_layer3(na, nbp, da, db, b2r, w3c,
                              w3c * a_src3[0], w3c * a_dst3[0])
    num3, den3 = _sc_edge3(h3.reshape(NP), as3.reshape(NP), ad3.reshape(NP),
                           src_t, dst_t, z1d)

    out = _tc_logsoftmax(num3.reshape(2, NP), den3.reshape(2, NP))
    return out[:, :N]


# speedup vs baseline: 38.1956x; 1.2169x over previous
import functools

import jax
import jax.numpy as jnp
from jax import lax
from jax.experimental import pallas as pl
from jax.experimental.pallas import tpu as pltpu
from jax.experimental.pallas import tpu_sc as plsc

N = 10000
E = 320000
D = 128
NP = 10240
NS = 16
K = 128
CH = 80
CB = 16
CH_TOT = 160
CH0 = 104
CH1 = CH_TOT - CH0
CAP = 32 * CH * K
ROWS_PT = NP // NS
F32 = jnp.float32


def _elu(g):
    return jnp.where(g > 0, g, jnp.exp(g) - 1.0)


def _tc_layer1_body(x_ref, w_ref, asrc_ref, adst_ref, h_ref, as_ref, ad_ref):
    h = jnp.dot(x_ref[...], w_ref[...], preferred_element_type=F32)
    h_ref[...] = h
    as_ref[...] = jnp.dot(h, asrc_ref[...], preferred_element_type=F32)
    ad_ref[...] = jnp.dot(h, adst_ref[...], preferred_element_type=F32)


def _tc_layer1(x, w, asrc, adst):
    grid = 8
    r = NP // grid
    return pl.pallas_call(
        _tc_layer1_body,
        grid=(grid,),
        in_specs=[
            pl.BlockSpec((r, D), lambda i: (i, 0)),
            pl.BlockSpec((D, D), lambda i: (0, 0)),
            pl.BlockSpec((D, 1), lambda i: (0, 0)),
            pl.BlockSpec((D, 1), lambda i: (0, 0)),
        ],
        out_specs=[
            pl.BlockSpec((r, D), lambda i: (i, 0)),
            pl.BlockSpec((r, 1), lambda i: (i, 0)),
            pl.BlockSpec((r, 1), lambda i: (i, 0)),
        ],
        out_shape=[
            jax.ShapeDtypeStruct((NP, D), F32),
            jax.ShapeDtypeStruct((NP, 1), F32),
            jax.ShapeDtypeStruct((NP, 1), F32),
        ],
    )(x, w, asrc, adst)


def _tc_layer2_body(na_ref, nb_ref, da_ref, db_ref, b_ref, w_ref, asrc_ref,
                    adst_ref, h_ref, as_ref, ad_ref):
    inv = 1.0 / (da_ref[...] + db_ref[...] + 1e-16)
    g = _elu((na_ref[...] + nb_ref[...]) * inv + b_ref[...])
    h = jnp.dot(g, w_ref[...], preferred_element_type=F32)
    h_ref[...] = h
    as_ref[...] = jnp.dot(h, asrc_ref[...], preferred_element_type=F32)
    ad_ref[...] = jnp.dot(h, adst_ref[...], preferred_element_type=F32)


def _tc_layer2(na, nb, da, db, b, w, asrc, adst):
    grid = 8
    r = NP // grid
    return pl.pallas_call(
        _tc_layer2_body,
        grid=(grid,),
        in_specs=[
            pl.BlockSpec((r, D), lambda i: (i, 0)),
            pl.BlockSpec((r, D), lambda i: (i, 0)),
            pl.BlockSpec((r, 1), lambda i: (i, 0)),
            pl.BlockSpec((r, 1), lambda i: (i, 0)),
            pl.BlockSpec((1, D), lambda i: (0, 0)),
            pl.BlockSpec((D, D), lambda i: (0, 0)),
            pl.BlockSpec((D, 1), lambda i: (0, 0)),
            pl.BlockSpec((D, 1), lambda i: (0, 0)),
        ],
        out_specs=[
            pl.BlockSpec((r, D), lambda i: (i, 0)),
            pl.BlockSpec((r, 1), lambda i: (i, 0)),
            pl.BlockSpec((r, 1), lambda i: (i, 0)),
        ],
        out_shape=[
            jax.ShapeDtypeStruct((NP, D), F32),
            jax.ShapeDtypeStruct((NP, 1), F32),
            jax.ShapeDtypeStruct((NP, 1), F32),
        ],
    )(na, nb, da, db, b, w, asrc, adst)


def _tc_layer3_body(na_ref, nb_ref, da_ref, db_ref, b_ref, w3_ref, ws_ref,
                    wd_ref, h3_ref, as_ref, ad_ref):
    inv = 1.0 / (da_ref[...] + db_ref[...] + 1e-16)
    g = _elu((na_ref[...] + nb_ref[...]) * inv + b_ref[...])
    h3_ref[...] = jnp.dot(g, w3_ref[...], preferred_element_type=F32)
    as_ref[...] = jnp.dot(g, ws_ref[...], preferred_element_type=F32)
    ad_ref[...] = jnp.dot(g, wd_ref[...], preferred_element_type=F32)


def _tc_layer3(na, nb, da, db, b, w3, ws, wd):
    grid = 8
    r = NP // grid
    return pl.pallas_call(
        _tc_layer3_body,
        grid=(grid,),
        in_specs=[
            pl.BlockSpec((r, D), lambda i: (i, 0)),
            pl.BlockSpec((r, D), lambda i: (i, 0)),
            pl.BlockSpec((r, 1), lambda i: (i, 0)),
            pl.BlockSpec((r, 1), lambda i: (i, 0)),
            pl.BlockSpec((1, D), lambda i: (0, 0)),
            pl.BlockSpec((D, 1), lambda i: (0, 0)),
            pl.BlockSpec((D, 1), lambda i: (0, 0)),
            pl.BlockSpec((D, 1), lambda i: (0, 0)),
        ],
        out_specs=[
            pl.BlockSpec((r, 1), lambda i: (i, 0)),
            pl.BlockSpec((r, 1), lambda i: (i, 0)),
            pl.BlockSpec((r, 1), lambda i: (i, 0)),
        ],
        out_shape=[
            jax.ShapeDtypeStruct((NP, 1), F32),
            jax.ShapeDtypeStruct((NP, 1), F32),
            jax.ShapeDtypeStruct((NP, 1), F32),
        ],
    )(na, nb, da, db, b, w3, ws, wd)


def _tc_logsoftmax_body(num_ref, den_ref, out_ref):
    n = num_ref[0:1, :] + num_ref[1:2, :]
    d = den_ref[0:1, :] + den_ref[1:2, :]
    l = n / (d + 1e-16)
    col = lax.broadcasted_iota(jnp.int32, (1, NP), 1)
    mask = col < N
    lm = jnp.where(mask, l, -3.0e38)
    m = jnp.max(lm)
    se = jnp.sum(jnp.where(mask, jnp.exp(lm - m), 0.0))
    out_ref[...] = (lm - m) - jnp.log(se)


def _tc_logsoftmax(num2, den2):
    return pl.pallas_call(
        _tc_logsoftmax_body,
        grid=(1,),
        in_specs=[
            pl.BlockSpec((2, NP), lambda i: (0, 0)),
            pl.BlockSpec((2, NP), lambda i: (0, 0)),
        ],
        out_specs=pl.BlockSpec((1, NP), lambda i: (0, 0)),
        out_shape=jax.ShapeDtypeStruct((1, NP), F32),
    )(num2, den2)


_MESH = plsc.VectorSubcoreMesh(core_axis_name="c", subcore_axis_name="s")


def _sc_edge12(h, asv, adv, src_t, dst_t, z2d, z1d):
    @functools.partial(
        pl.kernel,
        out_type=[
            jax.ShapeDtypeStruct((2 * NP, D), F32),
            jax.ShapeDtypeStruct((2 * NP,), F32),
        ],
        mesh=_MESH,
        compiler_params=pltpu.CompilerParams(needs_layout_passes=False),
        scratch_types=[
            pltpu.VMEM((CB, K), jnp.int32),
            pltpu.VMEM((CB, K), jnp.int32),
            pltpu.VMEM((K, D), F32),
            pltpu.VMEM((K, D), F32),
            pltpu.VMEM((K,), F32),
            pltpu.VMEM((K,), F32),
            pltpu.VMEM((K,), F32),
            pltpu.VMEM((K,), F32),
            pltpu.VMEM((K,), F32),
            pltpu.VMEM((K,), F32),
            pltpu.VMEM_SHARED((NP,), F32),
            pltpu.VMEM_SHARED((NP,), F32),
            pltpu.VMEM_SHARED((NP, D), F32),
            pltpu.VMEM_SHARED((NP,), F32),
            pltpu.SemaphoreType.DMA,
            pltpu.SemaphoreType.DMA,
            pltpu.SemaphoreType.DMA,
            pltpu.SemaphoreType.DMA,
            pltpu.SemaphoreType.DMA,
            pltpu.SemaphoreType.DMA,
            pltpu.SemaphoreType.DMA,
            pltpu.SemaphoreType.DMA,
        ],
    )
    def k(h_r, as_r, ad_r, src_r, dst_r, z2_r, z1_r,
          num_r, den_r,
          src_v, dst_v, rows_v0, rows_v1, asb0, asb1, adb0, adb1, exb0, exb1,
          as_sh, ad_sh, num_sh, den_sh,
          sem0, sem1, semA0, semA1, semD0, semD1, semS0, semS1):
        c = lax.axis_index("c")
        s = lax.axis_index("s")
        base = s * CH_TOT + c * CH0
        nblk = jnp.where(c == 0, CH0 // CB, CH1 // CB)
        r0 = s * ROWS_PT
        pltpu.sync_copy(z2_r.at[pl.ds(r0, ROWS_PT)],
                        num_sh.at[pl.ds(r0, ROWS_PT)])
        pltpu.sync_copy(z1_r.at[pl.ds(r0, ROWS_PT)],
                        den_sh.at[pl.ds(r0, ROWS_PT)])
        pltpu.sync_copy(as_r.at[pl.ds(r0, ROWS_PT)],
                        as_sh.at[pl.ds(r0, ROWS_PT)])
        pltpu.sync_copy(ad_r.at[pl.ds(r0, ROWS_PT)],
                        ad_sh.at[pl.ds(r0, ROWS_PT)])
        plsc.subcore_barrier()

        def fetch(i, rows_v, asb, adb, sem, semA):
            pltpu.async_copy(h_r.at[src_v.at[i]], rows_v, sem)
            pltpu.async_copy(as_sh.at[src_v.at[i]], asb, semA)
            pltpu.async_copy(ad_sh.at[dst_v.at[i]], adb, semA)

        def chunk(i, rows_v, asb, adb, exb, sem, semA, semD, semS,
                  rows_o, asb_o, adb_o, sem_o, semA_o, semS_o,
                  prefetch, pend_d, pend_s):
            @pl.when(pend_d)
            def _():
                pltpu.make_async_copy(exb, den_sh.at[dst_v.at[i]],
                                      semD).wait()
            pltpu.make_async_copy(as_sh.at[src_v.at[i]], asb, semA).wait()
            pltpu.make_async_copy(ad_sh.at[dst_v.at[i]], adb, semA).wait()
            for j in range(K // 16):
                e = asb[pl.ds(j * 16, 16)] + adb[pl.ds(j * 16, 16)]
                e = jnp.where(e >= 0, e, 0.2 * e)
                exb[pl.ds(j * 16, 16)] = jnp.exp(e)
            pltpu.async_copy(exb, den_sh.at[dst_v.at[i]], semD, add=True)

            @pl.when(pend_s)
            def _():
                pltpu.make_async_copy(rows_o, num_sh.at[dst_v.at[i]],
                                      semS_o).wait()

            @pl.when(prefetch)
            def _():
                fetch(i + 1, rows_o, asb_o, adb_o, sem_o, semA_o)

            pltpu.make_async_copy(h_r.at[src_v.at[i]], rows_v, sem).wait()
            for g in range(K // 16):
                exv = exb[pl.ds(g * 16, 16)]
                for jj in range(16):
                    cj = exv[jj]
                    r = g * 16 + jj
                    for v in range(D // 16):
                        rows_v[r, pl.ds(v * 16, 16)] = (
                            rows_v[r, pl.ds(v * 16, 16)] * cj)
            pltpu.async_copy(rows_v, num_sh.at[dst_v.at[i]], semS, add=True)

        def blk(bi, carry):
            pltpu.sync_copy(src_r.at[pl.ds(base + bi * CB, CB)], src_v)
            pltpu.sync_copy(dst_r.at[pl.ds(base + bi * CB, CB)], dst_v)
            fetch(0, rows_v0, asb0, adb0, sem0, semA0)

            def pair(g, c2):
                i = 2 * g
                ge1 = g >= 1
                chunk(i, rows_v0, asb0, adb0, exb0, sem0, semA0, semD0, semS0,
                      rows_v1, asb1, adb1, sem1, semA1, semS1,
                      True, ge1, ge1)
                chunk(i + 1, rows_v1, asb1, adb1, exb1, sem1, semA1, semD1,
                      semS1, rows_v0, asb0, adb0, sem0, semA0, semS0,
                      g < CB // 2 - 1, ge1, True)
                return c2

            lax.fori_loop(0, CB // 2, pair, 0)
            pltpu.make_async_copy(rows_v1, num_sh.at[dst_v.at[CB - 1]],
                                  semS1).wait()
            pltpu.make_async_copy(exb0, den_sh.at[dst_v.at[CB - 2]],
                                  semD0).wait()
            pltpu.make_async_copy(exb1, den_sh.at[dst_v.at[CB - 1]],
                                  semD1).wait()
            return carry

        lax.fori_loop(0, nblk, blk, 0)
        plsc.subcore_barrier()
        pltpu.sync_copy(num_sh.at[pl.ds(r0, ROWS_PT)],
                        num_r.at[pl.ds(c * NP + r0, ROWS_PT)])
        pltpu.sync_copy(den_sh.at[pl.ds(r0, ROWS_PT)],
                        den_r.at[pl.ds(c * NP + r0, ROWS_PT)])

    return k(h, asv, adv, src_t, dst_t, z2d, z1d)


def _sc_edge3(h3v, asv, adv, src_t, dst_t, z1d):
    @functools.partial(
        pl.kernel,
        out_type=[
            jax.ShapeDtypeStruct((2 * NP,), F32),
            jax.ShapeDtypeStruct((2 * NP,), F32),
        ],
        mesh=_MESH,
        compiler_params=pltpu.CompilerParams(needs_layout_passes=False),
        scratch_types=[
            pltpu.VMEM((CH, K), jnp.int32),
            pltpu.VMEM((CH, K), jnp.int32),
            pltpu.VMEM((NP,), F32),
            pltpu.VMEM((NP,), F32),
            pltpu.VMEM((NP,), F32),
            pltpu.VMEM((K,), F32),
            pltpu.VMEM((K,), F32),
            pltpu.VMEM_SHARED((NP,), F32),
            pltpu.VMEM_SHARED((NP,), F32),
        ],
    )
    def k(h3_r, as_r, ad_r, src_r, dst_r, z1_r,
          num_r, den_r,
          src_v, dst_v, h3_v, as_v, ad_v, exb, nb, num_sh, den_sh):
        c = lax.axis_index("c")
        s = lax.axis_index("s")
        w = c * NS + s
        r0 = s * ROWS_PT
        pltpu.sync_copy(z1_r.at[pl.ds(r0, ROWS_PT)],
                        num_sh.at[pl.ds(r0, ROWS_PT)])
        pltpu.sync_copy(z1_r.at[pl.ds(r0, ROWS_PT)],
                        den_sh.at[pl.ds(r0, ROWS_PT)])
        pltpu.sync_copy(src_r.at[pl.ds(w * CH, CH)], src_v)
        pltpu.sync_copy(dst_r.at[pl.ds(w * CH, CH)], dst_v)
        pltpu.sync_copy(h3_r, h3_v)
        pltpu.sync_copy(as_r, as_v)
        pltpu.sync_copy(ad_r, ad_v)
        plsc.subcore_barrier()

        def body(i, carry):
            for j in range(K // 16):
                sidx = src_v[i, pl.ds(j * 16, 16)]
                didx = dst_v[i, pl.ds(j * 16, 16)]
                av = plsc.load_gather(as_v, [sidx])
                dv = plsc.load_gather(ad_v, [didx])
                hv = plsc.load_gather(h3_v, [sidx])
                e = av + dv
                e = jnp.where(e >= 0, e, 0.2 * e)
                ex = jnp.exp(e)
                exb[pl.ds(j * 16, 16)] = ex
                nb[pl.ds(j * 16, 16)] = ex * hv
            pltpu.sync_copy(exb, den_sh.at[dst_v.at[i]], add=True)
            pltpu.sync_copy(nb, num_sh.at[dst_v.at[i]], add=True)
            return carry

        lax.fori_loop(0, CH, body, 0)
        plsc.subcore_barrier()
        pltpu.sync_copy(num_sh.at[pl.ds(r0, ROWS_PT)],
                        num_r.at[pl.ds(c * NP + r0, ROWS_PT)])
        pltpu.sync_copy(den_sh.at[pl.ds(r0, ROWS_PT)],
                        den_r.at[pl.ds(c * NP + r0, ROWS_PT)])

    return k(h3v, asv, adv, src_t, dst_t, z1d)


def kernel(x, edge_index, W1, a_src1, a_dst1, b1, W2, a_src2, a_dst2, b2,
           W3, a_src3, a_dst3, b3):
    src = edge_index[0].astype(jnp.int32)
    dst = edge_index[1].astype(jnp.int32)

    def pad_edges(v, fill):
        return jnp.concatenate(
            [v, jnp.full((CAP - E,), fill, jnp.int32)]).reshape(-1, K)

    src_t = pad_edges(src, 0)
    dst_t = pad_edges(dst, N)

    xp = jnp.pad(x, ((0, NP - N), (0, 0)))
    z2d = jnp.zeros((NP, D), F32)
    z1d = jnp.zeros((NP,), F32)

    def col(v):
        return v.reshape(D, 1)

    b1r = b1.reshape(1, D)
    b2r = b2.reshape(1, D)

    def split(num, den):
        num2 = num.reshape(2, NP, D)
        den2 = den.reshape(2, NP, 1)
        return num2[0], num2[1], den2[0], den2[1]

    h, asl, adl = _tc_layer1(xp, W1, col(a_src1), col(a_dst1))
    num, den = _sc_edge12(h, asl.reshape(NP), adl.reshape(NP),
                          src_t, dst_t, z2d, z1d)

    na, nbp, da, db = split(num, den)
    h, asl, adl = _tc_layer2(na, nbp, da, db, b1r, W2,
                             col(a_src2), col(a_dst2))
    num, den = _sc_edge12(h, asl.reshape(NP), adl.reshape(NP),
                          src_t, dst_t, z2d, z1d)

    na, nbp, da, db = split(num, den)
    w3c = W3.reshape(D, 1)
    h3, as3, ad3 = _tc_layer3(na, nbp, da, db, b2r, w3c,
                              w3c * a_src3[0], w3c * a_dst3[0])
    num3, den3 = _sc_edge3(h3.reshape(NP), as3.reshape(NP), ad3.reshape(NP),
                           src_t, dst_t, z1d)

    out = _tc_logsoftmax(num3.reshape(2, NP), den3.reshape(2, NP))
    return out[:, :N]
